# Initial kernel scaffold; baseline (speedup 1.0000x reference)
import jax
import jax.numpy as jnp
from jax.experimental import pallas as pl


def kernel(sentences, sent_lengths, table):
    raise NotImplementedError("write your pallas kernel here")


# speedup vs baseline: 1.2759x; 1.2759x over previous
import functools

import jax
import jax.numpy as jnp
from jax import lax
from jax.experimental import pallas as pl
from jax.experimental.pallas import tpu as pltpu
from jax.experimental.pallas import tpu_sc as plsc

_VOCAB = 1000000
_DIM = 32
_B = 4096
_L = 200
_N = _B * _L
_NW = 32
_SENT_PER_W = _B // _NW
_G = 16
_ROWS_PER_CHUNK = _G * _L
_SUB = _ROWS_PER_CHUNK // 128
_CHUNKS = _SENT_PER_W // _G


def kernel(sentences, sent_lengths, table):
    idx = sentences.reshape(_N)
    mesh = plsc.VectorSubcoreMesh(core_axis_name="c", subcore_axis_name="s")

    @functools.partial(
        pl.kernel,
        out_type=jax.ShapeDtypeStruct((_N, _DIM), jnp.float32),
        mesh=mesh,
        scratch_types=[
            pltpu.VMEM((_ROWS_PER_CHUNK,), jnp.int32),
            pltpu.VMEM((_ROWS_PER_CHUNK, _DIM), jnp.float32),
            pltpu.VMEM((_SENT_PER_W,), jnp.int32),
            pltpu.SemaphoreType.DMA,
        ],
        compiler_params=pltpu.CompilerParams(use_tc_tiling_on_sc=False),
    )
    def emb(idx_hbm, len_hbm, table_hbm, out_hbm, idx_v, rows_v, len_s, sem):
        wid = lax.axis_index("s") * 2 + lax.axis_index("c")
        pltpu.async_copy(
            len_hbm.at[pl.ds(wid * _SENT_PER_W, _SENT_PER_W)], len_s, sem
        ).wait()
        zeros16 = jnp.zeros((16,), jnp.float32)

        @pl.loop(0, _CHUNKS)
        def _(ci):
            row_base = (wid * _CHUNKS + ci) * _ROWS_PER_CHUNK
            pltpu.async_copy(
                idx_hbm.at[pl.ds(row_base, _ROWS_PER_CHUNK)], idx_v, sem
            ).wait()

            @pl.loop(0, _SUB)
            def _(j):
                pltpu.async_copy(
                    table_hbm.at[idx_v.at[pl.ds(j * 128, 128)]],
                    rows_v.at[pl.ds(j * 128, 128)],
                    sem,
                ).wait()

            lens = len_s[pl.ds(ci * _G, _G)]
            for s in range(_G):
                slen = lens[s]

                def _zero_row(r, carry, s=s):
                    rowid = s * _L + r
                    rows_v[rowid, pl.ds(0, 16)] = zeros16
                    rows_v[rowid, pl.ds(16, 16)] = zeros16
                    return carry

                lax.fori_loop(slen, _L, _zero_row, 0)

            pltpu.async_copy(
                rows_v, out_hbm.at[pl.ds(row_base, _ROWS_PER_CHUNK)], sem
            ).wait()

    out = emb(idx, sent_lengths, table)
    return out.reshape(_B, _L, _DIM)


# speedup vs baseline: 1.4183x; 1.1116x over previous
import functools

import jax
import jax.numpy as jnp
from jax import lax
from jax.experimental import pallas as pl
from jax.experimental.pallas import tpu as pltpu
from jax.experimental.pallas import tpu_sc as plsc

_VOCAB = 1000000
_DIM = 32
_B = 4096
_L = 200
_N = _B * _L
_NW = 32
_SENT_PER_W = _B // _NW
_G = 16
_ROWS_PER_CHUNK = _G * _L
_SUB = _ROWS_PER_CHUNK // 128
_CHUNKS = _SENT_PER_W // _G


def kernel(sentences, sent_lengths, table):
    idx = sentences.reshape(_N)
    mesh = plsc.VectorSubcoreMesh(core_axis_name="c", subcore_axis_name="s")

    @functools.partial(
        pl.kernel,
        out_type=jax.ShapeDtypeStruct((_N, _DIM), jnp.float32),
        mesh=mesh,
        scratch_types=[
            pltpu.VMEM((_ROWS_PER_CHUNK,), jnp.int32),
            pltpu.VMEM((_ROWS_PER_CHUNK, _DIM), jnp.float32),
            pltpu.VMEM((_SENT_PER_W,), jnp.int32),
            pltpu.SemaphoreType.DMA,
        ],
        compiler_params=pltpu.CompilerParams(use_tc_tiling_on_sc=False),
    )
    def emb(idx_hbm, len_hbm, table_hbm, out_hbm, idx_v, rows_v, len_s, sem):
        wid = lax.axis_index("s") * 2 + lax.axis_index("c")
        pltpu.async_copy(
            len_hbm.at[pl.ds(wid * _SENT_PER_W, _SENT_PER_W)], len_s, sem
        ).wait()
        zeros16 = jnp.zeros((16,), jnp.float32)

        @pl.loop(0, _CHUNKS)
        def _(ci):
            row_base = (wid * _CHUNKS + ci) * _ROWS_PER_CHUNK
            pltpu.async_copy(
                idx_hbm.at[pl.ds(row_base, _ROWS_PER_CHUNK)], idx_v, sem
            ).wait()

            pltpu.async_copy(table_hbm.at[idx_v], rows_v, sem).wait()

            lens = len_s[pl.ds(ci * _G, _G)]
            for s in range(_G):
                slen = lens[s]

                def _zero_row(r, carry, s=s):
                    rowid = s * _L + r
                    rows_v[rowid, pl.ds(0, 16)] = zeros16
                    rows_v[rowid, pl.ds(16, 16)] = zeros16
                    return carry

                lax.fori_loop(slen, _L, _zero_row, 0)

            pltpu.async_copy(
                rows_v, out_hbm.at[pl.ds(row_base, _ROWS_PER_CHUNK)], sem
            ).wait()

    out = emb(idx, sent_lengths, table)
    return out.reshape(_B, _L, _DIM)


# speedup vs baseline: 1.4709x; 1.0371x over previous
import functools

import jax
import jax.numpy as jnp
from jax import lax
from jax.experimental import pallas as pl
from jax.experimental.pallas import tpu as pltpu
from jax.experimental.pallas import tpu_sc as plsc

_VOCAB = 1000000
_DIM = 32
_B = 4096
_L = 200
_N = _B * _L
_NW = 32
_SENT_PER_W = _B // _NW
_G = 8
_ROWS = _G * _L
_CHUNKS = _SENT_PER_W // _G


def kernel(sentences, sent_lengths, table):
    idx = sentences.reshape(_N)
    mesh = plsc.VectorSubcoreMesh(core_axis_name="c", subcore_axis_name="s")

    @functools.partial(
        pl.kernel,
        out_type=jax.ShapeDtypeStruct((_N, _DIM), jnp.float32),
        mesh=mesh,
        scratch_types=[
            pltpu.VMEM((2, _ROWS), jnp.int32),
            pltpu.VMEM((_ROWS, _DIM), jnp.float32),
            pltpu.VMEM((_ROWS, _DIM), jnp.float32),
            pltpu.VMEM((_SENT_PER_W + 32,), jnp.int32),
            pltpu.SemaphoreType.DMA,
            pltpu.SemaphoreType.DMA,
            pltpu.SemaphoreType.DMA,
            pltpu.SemaphoreType.DMA,
            pltpu.SemaphoreType.DMA,
            pltpu.SemaphoreType.DMA,
            pltpu.SemaphoreType.DMA,
        ],
        compiler_params=pltpu.CompilerParams(use_tc_tiling_on_sc=False),
    )
    def emb(idx_hbm, len_hbm, table_hbm, out_hbm, idx_v, rows0, rows1,
            len_s, sem_l, si0, si1, sg0, sg1, so0, so1):
        rows = (rows0, rows1)
        sem_i = (si0, si1)
        sem_g = (sg0, sg1)
        sem_o = (so0, so1)
        wid = lax.axis_index("s") * 2 + lax.axis_index("c")
        pltpu.async_copy(
            len_hbm.at[pl.ds(wid * _SENT_PER_W, _SENT_PER_W)],
            len_s.at[pl.ds(0, _SENT_PER_W)],
            sem_l,
        ).wait()
        zeros16 = jnp.zeros((16,), jnp.float32)
        base = wid * _CHUNKS * _ROWS

        def start_idx(b, c):
            return pltpu.async_copy(
                idx_hbm.at[pl.ds(base + c * _ROWS, _ROWS)],
                idx_v.at[b],
                sem_i[b],
            )

        def start_gather(b):
            return pltpu.async_copy(
                table_hbm.at[idx_v.at[b]], rows[b], sem_g[b]
            )

        def start_out(b, c):
            return pltpu.async_copy(
                rows[b], out_hbm.at[pl.ds(base + c * _ROWS, _ROWS)], sem_o[b]
            )

        idx_h = [start_idx(0, 0), start_idx(1, 1)]
        idx_h[0].wait()
        g_h = [start_gather(0), None]
        out_h = [None, None]

        for c in range(_CHUNKS):
            b = c & 1
            g_h[b].wait()
            if c + 1 < _CHUNKS:
                o = b ^ 1
                if out_h[o] is not None:
                    out_h[o].wait()
                idx_h[o].wait()
                g_h[o] = start_gather(o)
            if c + 2 < _CHUNKS:
                idx_h[b] = start_idx(b, c + 2)
            lens = len_s[pl.ds(c * _G, 16)]
            for s in range(_G):
                slen = lens[s]

                def _zero_row(r, carry, s=s, b=b):
                    rowid = s * _L + r
                    rows[b][rowid, pl.ds(0, 16)] = zeros16
                    rows[b][rowid, pl.ds(16, 16)] = zeros16
                    return carry

                lax.fori_loop(slen, _L, _zero_row, 0)
            out_h[b] = start_out(b, c)

        out_h[0].wait()
        out_h[1].wait()

    out = emb(idx, sent_lengths, table)
    return out.reshape(_B, _L, _DIM)
